# baseline (speedup 1.0000x reference)
import functools

import jax
import jax.numpy as jnp
from jax import lax
from jax.experimental import pallas as pl
from jax.experimental.pallas import tpu as pltpu
from jax.experimental.pallas import tpu_sc as plsc

F = 128
TCOLS = 512
BE = 16
SIDE = 5 * BE
RPW = 4
NWORK = 32


def _node_table_body(q_ref, mu_ref, w1_ref, b1_ref, w2_ref, b2_ref,
                     cw1_ref, cb1_ref, cw2_ref, cb2_ref, t_ref, wv_ref):
    qb = q_ref[...]
    h = qb @ w1_ref[...] + b1_ref[...]
    h = h / (1.0 + jnp.exp(-h))
    x = h @ w2_ref[...] + b2_ref[...]
    mu0 = mu_ref[:, 0:256]
    mu1 = mu_ref[:, 256:512]
    mu2 = mu_ref[:, 512:768]
    si = jnp.sqrt(mu0 * mu0 + mu1 * mu1 + mu2 * mu2)
    hh = jnp.maximum(si @ cw1_ref[...] + cb1_ref[...], 0.0)
    logits = hh @ cw2_ref[...] + cb2_ref[...]
    m = jnp.max(logits, axis=-1, keepdims=True)
    e = jnp.exp(logits - m)
    w = e / jnp.sum(e, axis=-1, keepdims=True)
    bn = qb.shape[0]
    v0 = jnp.sum(mu0, axis=-1, keepdims=True)
    v1 = jnp.sum(mu1, axis=-1, keepdims=True)
    v2 = jnp.sum(mu2, axis=-1, keepdims=True)
    t_ref[:, 0:256] = x[:, 0:256]
    t_ref[:, 256:384] = x[:, 256:384] * w
    t_ref[:, 384:512] = jnp.concatenate(
        [v0, v1, v2, jnp.zeros((bn, 125), jnp.float32)], axis=-1)
    wv_ref[:, 0:128] = w * v0
    wv_ref[:, 128:256] = w * v1
    wv_ref[:, 256:384] = w * v2


def _node_table(q2, mu2, W1, b1, W2, b2, Cw1, Cb1, Cw2, Cb2):
    n = q2.shape[0]
    bn = 400
    grid = n // bn
    full = lambda shape: pl.BlockSpec(shape, lambda i: (0, 0))
    return pl.pallas_call(
        _node_table_body,
        grid=(grid,),
        in_specs=[
            pl.BlockSpec((bn, F), lambda i: (i, 0)),
            pl.BlockSpec((bn, 768), lambda i: (i, 0)),
            full((F, F)), full((1, F)), full((F, 384)), full((1, 384)),
            full((256, 32)), full((1, 32)), full((32, F)), full((1, F)),
        ],
        out_specs=[pl.BlockSpec((bn, TCOLS), lambda i: (i, 0)),
                   pl.BlockSpec((bn, 384), lambda i: (i, 0))],
        out_shape=[jax.ShapeDtypeStruct((n, TCOLS), jnp.float32),
                   jax.ShapeDtypeStruct((n, 384), jnp.float32)],
    )(q2, mu2, W1, b1.reshape(1, -1), W2, b2.reshape(1, -1),
      Cw1, Cb1.reshape(1, -1), Cw2, Cb2.reshape(1, -1))


def _edge_body(nw,
               t_hbm, wij_hbm, side_hbm, bt_hbm, out_hbm,
               side0_v, side1_v, wij0_v, wij1_v, trows0_v, trows1_v,
               bt_v, acc_v, sems, semw, semg, semo):
    cid = lax.axis_index("c")
    sid = lax.axis_index("s")
    w = cid * 16 + sid
    pltpu.sync_copy(bt_hbm, bt_v)
    bw = bt_v[w, pl.ds(0, 16)]
    cols = lax.iota(jnp.int32, 16)
    zero16 = jnp.zeros((16,), jnp.float32)
    acc_rows = acc_v.shape[0]
    sideb = (side0_v, side1_v)
    wijb = (wij0_v, wij1_v)
    trowsb = (trows0_v, trows1_v)

    def _bcast(v, lane):
        idx = jnp.full((16,), lane, jnp.int32)
        return v.at[idx].get(mode=lax.GatherScatterMode.PROMISE_IN_BOUNDS)

    def _issue_sw(slot, t):
        pltpu.async_copy(side_hbm.at[pl.ds(t * SIDE, SIDE)],
                         sideb[slot], sems.at[slot])
        pltpu.async_copy(wij_hbm.at[pl.ds(t * BE, BE)],
                         wijb[slot], semw.at[slot])

    def _wait_side(slot, t):
        pltpu.make_async_copy(side_hbm.at[pl.ds(t * SIDE, SIDE)],
                              sideb[slot], sems.at[slot]).wait()

    def _wait_wij(slot, t):
        pltpu.make_async_copy(wij_hbm.at[pl.ds(t * BE, BE)],
                              wijb[slot], semw.at[slot]).wait()

    def _issue_gather(slot):
        pltpu.async_copy(t_hbm.at[sideb[slot].at[pl.ds(BE, BE)]],
                         trowsb[slot], semg.at[slot])

    def _wait_gather(slot):
        pltpu.make_async_copy(t_hbm.at[sideb[slot].at[pl.ds(BE, BE)]],
                              trowsb[slot], semg.at[slot]).wait()

    def _compute(slot, node_base):
        side_v = sideb[slot]
        wij_v = wijb[slot]
        trows_v = trowsb[slot]
        iv = side_v[pl.ds(0, 16)]
        valid = (iv >= node_base) & (iv < node_base + nw)
        loc16 = jnp.where(valid, iv - node_base, nw)
        d016 = plsc.bitcast(side_v[pl.ds(2 * BE, 16)], jnp.float32)
        d116 = plsc.bitcast(side_v[pl.ds(3 * BE, 16)], jnp.float32)
        d216 = plsc.bitcast(side_v[pl.ds(4 * BE, 16)], jnp.float32)

        def _edge4(i, _):
            for eu in range(4):
                e = i * 4 + eu
                locb = _bcast(loc16, e)
                d0 = _bcast(d016, e)
                d1 = _bcast(d116, e)
                d2 = _bcast(d216, e)
                vrow = trows_v[e, pl.ds(384, 16)]
                v0 = _bcast(vrow, 0)
                v1 = _bcast(vrow, 1)
                v2 = _bcast(vrow, 2)
                dd = (d0, d1, d2)
                vv = (v0, v1, v2)
                for f in range(F // 16):
                    s = f * 16
                    xv = trows_v[e, pl.ds(s, 16)] * wij_v[e, pl.ds(s, 16)]
                    plsc.addupdate_scatter(acc_v, [locb, cols + s], xv)
                    t1v = trows_v[e, pl.ds(128 + s, 16)] * wij_v[e, pl.ds(128 + s, 16)]
                    t2v = trows_v[e, pl.ds(256 + s, 16)] * wij_v[e, pl.ds(256 + s, 16)]
                    for c in range(3):
                        mv = t1v * dd[c] + t2v * vv[c]
                        plsc.addupdate_scatter(
                            acc_v, [locb, cols + (128 * (c + 1) + s)], mv)
            return 0

        lax.fori_loop(0, BE // 4, _edge4, 0)

    for k in range(RPW):
        node_base = (w * RPW + k) * nw
        e_start = bw[k]
        e_end = bw[k + 1]

        def _zr(r, _):
            for cc in range(512 // 16):
                acc_v[r, pl.ds(cc * 16, 16)] = zero16
            return 0

        lax.fori_loop(0, acc_rows, _zr, 0)

        t0 = e_start // BE
        t1 = (e_end + BE - 1) // BE
        nb = t1 - t0

        @pl.when(nb > 0)
        def _():
            _issue_sw(0, t0)
            _wait_side(0, t0)
            _issue_gather(0)

        @pl.when(nb > 1)
        def _():
            _issue_sw(1, t0 + 1)

        def _pair(p, _):
            i0 = 2 * p
            i1 = 2 * p + 1

            @pl.when(i1 < nb)
            def _():
                _wait_side(1, t0 + i1)
                _issue_gather(1)

            _wait_gather(0)
            _wait_wij(0, t0 + i0)
            _compute(0, node_base)

            @pl.when(i0 + 2 < nb)
            def _():
                _issue_sw(0, t0 + i0 + 2)
                _wait_side(0, t0 + i0 + 2)
                _issue_gather(0)

            @pl.when(i1 < nb)
            def _():
                _wait_gather(1)
                _wait_wij(1, t0 + i1)
                _compute(1, node_base)

                @pl.when(i1 + 2 < nb)
                def _():
                    _issue_sw(1, t0 + i1 + 2)

            return 0

        lax.fori_loop(0, (nb + 1) // 2, _pair, 0)

        pltpu.async_copy(acc_v.at[pl.ds(0, nw)],
                         out_hbm.at[pl.ds(node_base, nw)], semo).wait()


def _edge_phase(t_tab, wij2, side, bt, nw):
    n_pad = NWORK * RPW * nw
    mesh = plsc.VectorSubcoreMesh(core_axis_name="c", subcore_axis_name="s")
    body = functools.partial(_edge_body, nw)
    f = pl.kernel(
        body,
        out_type=jax.ShapeDtypeStruct((n_pad, 512), jnp.float32),
        mesh=mesh,
        scratch_types=[
            pltpu.VMEM((SIDE,), jnp.int32),
            pltpu.VMEM((SIDE,), jnp.int32),
            pltpu.VMEM((BE, 384), jnp.float32),
            pltpu.VMEM((BE, 384), jnp.float32),
            pltpu.VMEM((BE, TCOLS), jnp.float32),
            pltpu.VMEM((BE, TCOLS), jnp.float32),
            pltpu.VMEM((NWORK, 16), jnp.int32),
            pltpu.VMEM((nw + 8, 512), jnp.float32),
            pltpu.SemaphoreType.DMA((2,)),
            pltpu.SemaphoreType.DMA((2,)),
            pltpu.SemaphoreType.DMA((2,)),
            pltpu.SemaphoreType.DMA,
        ],
        compiler_params=pltpu.CompilerParams(needs_layout_passes=False),
    )
    return f(t_tab, wij2, side, bt)


def _post_body(q_ref, acc_ref, wv_ref, rw1_ref, rb1_ref, rw2_ref, rb2_ref,
               qo_ref, muo_ref, dtm_ref):
    dq = acc_ref[:, 0:128]
    qo_ref[...] = q_ref[...] + dq
    m = []
    for c in range(3):
        mc = acc_ref[:, 128 * (c + 1):128 * (c + 2)] + wv_ref[:, 128 * c:128 * (c + 1)]
        muo_ref[:, 128 * c:128 * (c + 1)] = mc
        m.append(mc)
    si = jnp.sqrt(m[0] * m[0] + m[1] * m[1] + m[2] * m[2])
    h = jnp.maximum(si @ rw1_ref[...] + rb1_ref[...], 0.0)
    sw = h @ rw2_ref[...] + rb2_ref[...]
    bn = h.shape[0]
    sw3 = sw.reshape(bn, F, F)
    ones = jnp.ones((F, 1), jnp.float32)
    for i in range(3):
        for j in range(i, 3):
            r2 = m[i] * m[j]
            y = (sw3 * r2[:, None, :]).reshape(bn * F, F) @ ones
            d = y.reshape(bn, F)
            dtm_ref[:, 3 * i + j, :] = d
            if i != j:
                dtm_ref[:, 3 * j + i, :] = d


def _post_phase(q2, acc, wv, Rw1, Rb1, Rw2, Rb2):
    n = q2.shape[0]
    bn = 200
    grid = n // bn
    full = lambda shape: pl.BlockSpec(shape, lambda i: (0, 0))
    return pl.pallas_call(
        _post_body,
        grid=(grid,),
        in_specs=[
            pl.BlockSpec((bn, F), lambda i: (i, 0)),
            pl.BlockSpec((bn, 512), lambda i: (i, 0)),
            pl.BlockSpec((bn, 384), lambda i: (i, 0)),
            full((F, 32)), full((1, 32)), full((32, F * F)), full((1, F * F)),
        ],
        out_specs=[
            pl.BlockSpec((bn, F), lambda i: (i, 0)),
            pl.BlockSpec((bn, 384), lambda i: (i, 0)),
            pl.BlockSpec((bn, 9, F), lambda i: (i, 0, 0)),
        ],
        out_shape=[
            jax.ShapeDtypeStruct((n, F), jnp.float32),
            jax.ShapeDtypeStruct((n, 384), jnp.float32),
            jax.ShapeDtypeStruct((n, 9, F), jnp.float32),
        ],
    )(q2, acc, wv, Rw1, Rb1.reshape(1, -1), Rw2, Rb2.reshape(1, -1))


def kernel(q, mu, Wij, dir_ij, idx_i, idx_j, n_atoms, W1, b1, W2, b2,
           Cw1, Cb1, Cw2, Cb2, Rw1, Rb1, Rw2, Rb2):
    n = q.shape[0]
    e_total = idx_i.shape[0]
    q2 = q.reshape(n, F)
    mu2 = mu.reshape(n, 768)
    wij2 = Wij.reshape(e_total, 384)

    dbits = lax.bitcast_convert_type(dir_ij.T, jnp.int32)
    side = jnp.concatenate([idx_i.reshape(1, -1), idx_j.reshape(1, -1), dbits], axis=0)
    side = side.reshape(5, e_total // BE, BE).transpose(1, 0, 2).reshape(-1)

    nranges = NWORK * RPW
    nw = ((-(-n // nranges) + 7) // 8) * 8
    marks = jnp.arange(nranges + 1, dtype=jnp.int32) * nw
    ss = jnp.searchsorted(idx_i, marks).astype(jnp.int32)
    sel = jnp.arange(NWORK)[:, None] * RPW + jnp.arange(RPW + 1)[None, :]
    bt = jnp.zeros((NWORK, 16), jnp.int32)
    bt = bt.at[:, :RPW + 1].set(ss[sel])

    t_tab, wv = _node_table(q2, mu2, W1, b1, W2, b2, Cw1, Cb1, Cw2, Cb2)
    acc = _edge_phase(t_tab, wij2, side, bt, nw)
    q_out, mu_new, dtm9 = _post_phase(q2, acc, wv, Rw1, Rb1, Rw2, Rb2)
    return (q_out.reshape(n, 1, F),
            mu_new.reshape(n, 3, F),
            dtm9.transpose(0, 2, 1).reshape(n, F, 3, 3))


# speedup vs baseline: 1.0402x; 1.0402x over previous
import functools

import jax
import jax.numpy as jnp
from jax import lax
from jax.experimental import pallas as pl
from jax.experimental.pallas import tpu as pltpu
from jax.experimental.pallas import tpu_sc as plsc

F = 128
TCOLS = 512
BE = 32
SIDE = 5 * BE
RPW = 4
NWORK = 32


def _node_table_body(q_ref, mu_ref, w1_ref, b1_ref, w2_ref, b2_ref,
                     cw1_ref, cb1_ref, cw2_ref, cb2_ref, t_ref, wv_ref):
    qb = q_ref[...]
    h = qb @ w1_ref[...] + b1_ref[...]
    h = h / (1.0 + jnp.exp(-h))
    x = h @ w2_ref[...] + b2_ref[...]
    mu0 = mu_ref[:, 0:256]
    mu1 = mu_ref[:, 256:512]
    mu2 = mu_ref[:, 512:768]
    si = jnp.sqrt(mu0 * mu0 + mu1 * mu1 + mu2 * mu2)
    hh = jnp.maximum(si @ cw1_ref[...] + cb1_ref[...], 0.0)
    logits = hh @ cw2_ref[...] + cb2_ref[...]
    m = jnp.max(logits, axis=-1, keepdims=True)
    e = jnp.exp(logits - m)
    w = e / jnp.sum(e, axis=-1, keepdims=True)
    bn = qb.shape[0]
    v0 = jnp.sum(mu0, axis=-1, keepdims=True)
    v1 = jnp.sum(mu1, axis=-1, keepdims=True)
    v2 = jnp.sum(mu2, axis=-1, keepdims=True)
    t_ref[:, 0:256] = x[:, 0:256]
    t_ref[:, 256:384] = x[:, 256:384] * w
    t_ref[:, 384:512] = jnp.concatenate(
        [v0, v1, v2, jnp.zeros((bn, 125), jnp.float32)], axis=-1)
    wv_ref[:, 0:128] = w * v0
    wv_ref[:, 128:256] = w * v1
    wv_ref[:, 256:384] = w * v2


def _node_table(q2, mu2, W1, b1, W2, b2, Cw1, Cb1, Cw2, Cb2):
    n = q2.shape[0]
    bn = 400
    grid = n // bn
    full = lambda shape: pl.BlockSpec(shape, lambda i: (0, 0))
    return pl.pallas_call(
        _node_table_body,
        grid=(grid,),
        in_specs=[
            pl.BlockSpec((bn, F), lambda i: (i, 0)),
            pl.BlockSpec((bn, 768), lambda i: (i, 0)),
            full((F, F)), full((1, F)), full((F, 384)), full((1, 384)),
            full((256, 32)), full((1, 32)), full((32, F)), full((1, F)),
        ],
        out_specs=[pl.BlockSpec((bn, TCOLS), lambda i: (i, 0)),
                   pl.BlockSpec((bn, 384), lambda i: (i, 0))],
        out_shape=[jax.ShapeDtypeStruct((n, TCOLS), jnp.float32),
                   jax.ShapeDtypeStruct((n, 384), jnp.float32)],
    )(q2, mu2, W1, b1.reshape(1, -1), W2, b2.reshape(1, -1),
      Cw1, Cb1.reshape(1, -1), Cw2, Cb2.reshape(1, -1))


def _edge_body(nw,
               t_hbm, wij_hbm, side_hbm, bt_hbm, out_hbm,
               side0_v, side1_v, wij0_v, wij1_v, trows0_v, trows1_v,
               bt_v, acc_v, sems, semw, semg, semo):
    cid = lax.axis_index("c")
    sid = lax.axis_index("s")
    w = cid * 16 + sid
    pltpu.sync_copy(bt_hbm, bt_v)
    bw = bt_v[w, pl.ds(0, 16)]
    cols = lax.iota(jnp.int32, 16)
    zero16 = jnp.zeros((16,), jnp.float32)
    acc_rows = acc_v.shape[0]
    sideb = (side0_v, side1_v)
    wijb = (wij0_v, wij1_v)
    trowsb = (trows0_v, trows1_v)

    def _bcast(v, lane):
        idx = jnp.full((16,), lane, jnp.int32)
        return v.at[idx].get(mode=lax.GatherScatterMode.PROMISE_IN_BOUNDS)

    def _issue_sw(slot, t):
        pltpu.async_copy(side_hbm.at[pl.ds(t * SIDE, SIDE)],
                         sideb[slot], sems.at[slot])
        pltpu.async_copy(wij_hbm.at[pl.ds(t * BE, BE)],
                         wijb[slot], semw.at[slot])

    def _wait_side(slot, t):
        pltpu.make_async_copy(side_hbm.at[pl.ds(t * SIDE, SIDE)],
                              sideb[slot], sems.at[slot]).wait()

    def _wait_wij(slot, t):
        pltpu.make_async_copy(wij_hbm.at[pl.ds(t * BE, BE)],
                              wijb[slot], semw.at[slot]).wait()

    def _issue_gather(slot):
        pltpu.async_copy(t_hbm.at[sideb[slot].at[pl.ds(BE, BE)]],
                         trowsb[slot], semg.at[slot])

    def _wait_gather(slot):
        pltpu.make_async_copy(t_hbm.at[sideb[slot].at[pl.ds(BE, BE)]],
                              trowsb[slot], semg.at[slot]).wait()

    def _compute(slot, node_base):
        side_v = sideb[slot]
        wij_v = wijb[slot]
        trows_v = trowsb[slot]
        for vc in range(BE // 16):
            vb = vc * 16
            iv = side_v[pl.ds(vb, 16)]
            valid = (iv >= node_base) & (iv < node_base + nw)
            loc16 = jnp.where(valid, iv - node_base, nw)
            d016 = plsc.bitcast(side_v[pl.ds(2 * BE + vb, 16)], jnp.float32)
            d116 = plsc.bitcast(side_v[pl.ds(3 * BE + vb, 16)], jnp.float32)
            d216 = plsc.bitcast(side_v[pl.ds(4 * BE + vb, 16)], jnp.float32)

            def _edge4(i, _):
                for eu in range(2):
                    e16 = i * 2 + eu
                    e = vb + e16
                    locb = _bcast(loc16, e16)
                    d0 = _bcast(d016, e16)
                    d1 = _bcast(d116, e16)
                    d2 = _bcast(d216, e16)
                    vrow = trows_v[e, pl.ds(384, 16)]
                    v0 = _bcast(vrow, 0)
                    v1 = _bcast(vrow, 1)
                    v2 = _bcast(vrow, 2)
                    dd = (d0, d1, d2)
                    vv = (v0, v1, v2)
                    for f in range(F // 16):
                        s = f * 16
                        xv = trows_v[e, pl.ds(s, 16)] * wij_v[e, pl.ds(s, 16)]
                        plsc.addupdate_scatter(acc_v, [locb, cols + s], xv)
                        t1v = trows_v[e, pl.ds(128 + s, 16)] * wij_v[e, pl.ds(128 + s, 16)]
                        t2v = trows_v[e, pl.ds(256 + s, 16)] * wij_v[e, pl.ds(256 + s, 16)]
                        for c in range(3):
                            mv = t1v * dd[c] + t2v * vv[c]
                            plsc.addupdate_scatter(
                                acc_v, [locb, cols + (128 * (c + 1) + s)], mv)
                return 0

            lax.fori_loop(0, 8, _edge4, 0)

    for k in range(RPW):
        node_base = (w * RPW + k) * nw
        e_start = bw[k]
        e_end = bw[k + 1]

        def _zr(r, _):
            for cc in range(512 // 16):
                acc_v[r, pl.ds(cc * 16, 16)] = zero16
            return 0

        lax.fori_loop(0, acc_rows, _zr, 0)

        t0 = e_start // BE
        t1 = (e_end + BE - 1) // BE
        nb = t1 - t0

        @pl.when(nb > 0)
        def _():
            _issue_sw(0, t0)
            _wait_side(0, t0)
            _issue_gather(0)

        @pl.when(nb > 1)
        def _():
            _issue_sw(1, t0 + 1)

        def _pair(p, _):
            i0 = 2 * p
            i1 = 2 * p + 1

            @pl.when(i1 < nb)
            def _():
                _wait_side(1, t0 + i1)
                _issue_gather(1)

            _wait_gather(0)
            _wait_wij(0, t0 + i0)
            _compute(0, node_base)

            @pl.when(i0 + 2 < nb)
            def _():
                _issue_sw(0, t0 + i0 + 2)
                _wait_side(0, t0 + i0 + 2)
                _issue_gather(0)

            @pl.when(i1 < nb)
            def _():
                _wait_gather(1)
                _wait_wij(1, t0 + i1)
                _compute(1, node_base)

                @pl.when(i1 + 2 < nb)
                def _():
                    _issue_sw(1, t0 + i1 + 2)

            return 0

        lax.fori_loop(0, (nb + 1) // 2, _pair, 0)

        pltpu.async_copy(acc_v.at[pl.ds(0, nw)],
                         out_hbm.at[pl.ds(node_base, nw)], semo).wait()


def _edge_phase(t_tab, wij2, side, bt, nw):
    n_pad = NWORK * RPW * nw
    mesh = plsc.VectorSubcoreMesh(core_axis_name="c", subcore_axis_name="s")
    body = functools.partial(_edge_body, nw)
    f = pl.kernel(
        body,
        out_type=jax.ShapeDtypeStruct((n_pad, 512), jnp.float32),
        mesh=mesh,
        scratch_types=[
            pltpu.VMEM((SIDE,), jnp.int32),
            pltpu.VMEM((SIDE,), jnp.int32),
            pltpu.VMEM((BE, 384), jnp.float32),
            pltpu.VMEM((BE, 384), jnp.float32),
            pltpu.VMEM((BE, TCOLS), jnp.float32),
            pltpu.VMEM((BE, TCOLS), jnp.float32),
            pltpu.VMEM((NWORK, 16), jnp.int32),
            pltpu.VMEM((nw + 8, 512), jnp.float32),
            pltpu.SemaphoreType.DMA((2,)),
            pltpu.SemaphoreType.DMA((2,)),
            pltpu.SemaphoreType.DMA((2,)),
            pltpu.SemaphoreType.DMA,
        ],
        compiler_params=pltpu.CompilerParams(needs_layout_passes=False),
    )
    return f(t_tab, wij2, side, bt)


def _post_body(q_ref, acc_ref, wv_ref, rw1_ref, rb1_ref, rw2_ref, rb2_ref,
               qo_ref, muo_ref, dtm_ref):
    dq = acc_ref[:, 0:128]
    qo_ref[...] = q_ref[...] + dq
    m = []
    for c in range(3):
        mc = acc_ref[:, 128 * (c + 1):128 * (c + 2)] + wv_ref[:, 128 * c:128 * (c + 1)]
        muo_ref[:, 128 * c:128 * (c + 1)] = mc
        m.append(mc)
    si = jnp.sqrt(m[0] * m[0] + m[1] * m[1] + m[2] * m[2])
    h = jnp.maximum(si @ rw1_ref[...] + rb1_ref[...], 0.0)
    sw = h @ rw2_ref[...] + rb2_ref[...]
    bn = h.shape[0]
    sw3 = sw.reshape(bn, F, F)
    ones = jnp.ones((F, 1), jnp.float32)
    for i in range(3):
        for j in range(i, 3):
            r2 = m[i] * m[j]
            y = (sw3 * r2[:, None, :]).reshape(bn * F, F) @ ones
            d = y.reshape(bn, F)
            dtm_ref[:, 3 * i + j, :] = d
            if i != j:
                dtm_ref[:, 3 * j + i, :] = d


def _post_phase(q2, acc, wv, Rw1, Rb1, Rw2, Rb2):
    n = q2.shape[0]
    bn = 200
    grid = n // bn
    full = lambda shape: pl.BlockSpec(shape, lambda i: (0, 0))
    return pl.pallas_call(
        _post_body,
        grid=(grid,),
        in_specs=[
            pl.BlockSpec((bn, F), lambda i: (i, 0)),
            pl.BlockSpec((bn, 512), lambda i: (i, 0)),
            pl.BlockSpec((bn, 384), lambda i: (i, 0)),
            full((F, 32)), full((1, 32)), full((32, F * F)), full((1, F * F)),
        ],
        out_specs=[
            pl.BlockSpec((bn, F), lambda i: (i, 0)),
            pl.BlockSpec((bn, 384), lambda i: (i, 0)),
            pl.BlockSpec((bn, 9, F), lambda i: (i, 0, 0)),
        ],
        out_shape=[
            jax.ShapeDtypeStruct((n, F), jnp.float32),
            jax.ShapeDtypeStruct((n, 384), jnp.float32),
            jax.ShapeDtypeStruct((n, 9, F), jnp.float32),
        ],
    )(q2, acc, wv, Rw1, Rb1.reshape(1, -1), Rw2, Rb2.reshape(1, -1))


def kernel(q, mu, Wij, dir_ij, idx_i, idx_j, n_atoms, W1, b1, W2, b2,
           Cw1, Cb1, Cw2, Cb2, Rw1, Rb1, Rw2, Rb2):
    n = q.shape[0]
    e_total = idx_i.shape[0]
    q2 = q.reshape(n, F)
    mu2 = mu.reshape(n, 768)
    wij2 = Wij.reshape(e_total, 384)

    dbits = lax.bitcast_convert_type(dir_ij.T, jnp.int32)
    side = jnp.concatenate([idx_i.reshape(1, -1), idx_j.reshape(1, -1), dbits], axis=0)
    side = side.reshape(5, e_total // BE, BE).transpose(1, 0, 2).reshape(-1)

    nranges = NWORK * RPW
    nw = ((-(-n // nranges) + 7) // 8) * 8
    marks = jnp.arange(nranges + 1, dtype=jnp.int32) * nw
    ss = jnp.searchsorted(idx_i, marks).astype(jnp.int32)
    sel = jnp.arange(NWORK)[:, None] * RPW + jnp.arange(RPW + 1)[None, :]
    bt = jnp.zeros((NWORK, 16), jnp.int32)
    bt = bt.at[:, :RPW + 1].set(ss[sel])

    t_tab, wv = _node_table(q2, mu2, W1, b1, W2, b2, Cw1, Cb1, Cw2, Cb2)
    acc = _edge_phase(t_tab, wij2, side, bt, nw)
    q_out, mu_new, dtm9 = _post_phase(q2, acc, wv, Rw1, Rb1, Rw2, Rb2)
    return (q_out.reshape(n, 1, F),
            mu_new.reshape(n, 3, F),
            dtm9.transpose(0, 2, 1).reshape(n, F, 3, 3))


# speedup vs baseline: 1.7550x; 1.6872x over previous
import functools

import jax
import jax.numpy as jnp
from jax import lax
from jax.experimental import pallas as pl
from jax.experimental.pallas import tpu as pltpu
from jax.experimental.pallas import tpu_sc as plsc

F = 128
TCOLS = 512
BE = 32
SIDE = 5 * BE
RPW = 4
NWORK = 32


def _node_table_body(q_ref, mu_ref, w1_ref, b1_ref, w2_ref, b2_ref,
                     cw1_ref, cb1_ref, cw2_ref, cb2_ref, t_ref, wv_ref):
    qb = q_ref[...]
    h = qb @ w1_ref[...] + b1_ref[...]
    h = h / (1.0 + jnp.exp(-h))
    x = h @ w2_ref[...] + b2_ref[...]
    mu0 = mu_ref[:, 0:256]
    mu1 = mu_ref[:, 256:512]
    mu2 = mu_ref[:, 512:768]
    si = jnp.sqrt(mu0 * mu0 + mu1 * mu1 + mu2 * mu2)
    hh = jnp.maximum(si @ cw1_ref[...] + cb1_ref[...], 0.0)
    logits = hh @ cw2_ref[...] + cb2_ref[...]
    m = jnp.max(logits, axis=-1, keepdims=True)
    e = jnp.exp(logits - m)
    w = e / jnp.sum(e, axis=-1, keepdims=True)
    bn = qb.shape[0]
    v0 = jnp.sum(mu0, axis=-1, keepdims=True)
    v1 = jnp.sum(mu1, axis=-1, keepdims=True)
    v2 = jnp.sum(mu2, axis=-1, keepdims=True)
    t_ref[:, 0:256] = x[:, 0:256]
    t_ref[:, 256:384] = x[:, 256:384] * w
    t_ref[:, 384:512] = jnp.concatenate(
        [v0, v1, v2, jnp.zeros((bn, 125), jnp.float32)], axis=-1)
    wv_ref[:, 0:128] = w * v0
    wv_ref[:, 128:256] = w * v1
    wv_ref[:, 256:384] = w * v2


def _node_table(q2, mu2, W1, b1, W2, b2, Cw1, Cb1, Cw2, Cb2):
    n = q2.shape[0]
    bn = 400
    grid = n // bn
    full = lambda shape: pl.BlockSpec(shape, lambda i: (0, 0))
    return pl.pallas_call(
        _node_table_body,
        grid=(grid,),
        in_specs=[
            pl.BlockSpec((bn, F), lambda i: (i, 0)),
            pl.BlockSpec((bn, 768), lambda i: (i, 0)),
            full((F, F)), full((1, F)), full((F, 384)), full((1, 384)),
            full((256, 32)), full((1, 32)), full((32, F)), full((1, F)),
        ],
        out_specs=[pl.BlockSpec((bn, TCOLS), lambda i: (i, 0)),
                   pl.BlockSpec((bn, 384), lambda i: (i, 0))],
        out_shape=[jax.ShapeDtypeStruct((n, TCOLS), jnp.float32),
                   jax.ShapeDtypeStruct((n, 384), jnp.float32)],
    )(q2, mu2, W1, b1.reshape(1, -1), W2, b2.reshape(1, -1),
      Cw1, Cb1.reshape(1, -1), Cw2, Cb2.reshape(1, -1))


def _edge_body(nw,
               t_hbm, wij_hbm, side_hbm, bt_hbm, out_hbm,
               side0_v, side1_v, wij0_v, wij1_v, trows0_v, trows1_v,
               bt_v, acc_v, sems, semw, semg, semo):
    cid = lax.axis_index("c")
    sid = lax.axis_index("s")
    w = cid * 16 + sid
    pltpu.sync_copy(bt_hbm, bt_v)
    bw = bt_v[w, pl.ds(0, 16)]
    cols = lax.iota(jnp.int32, 16)
    zero16 = jnp.zeros((16,), jnp.float32)
    acc_rows = acc_v.shape[0]
    sideb = (side0_v, side1_v)
    wijb = (wij0_v, wij1_v)
    trowsb = (trows0_v, trows1_v)

    def _bcast(v, lane):
        idx = jnp.full((16,), lane, jnp.int32)
        return v.at[idx].get(mode=lax.GatherScatterMode.PROMISE_IN_BOUNDS)

    def _issue_sw(slot, t):
        pltpu.async_copy(side_hbm.at[pl.ds(t * SIDE, SIDE)],
                         sideb[slot], sems.at[slot])
        pltpu.async_copy(wij_hbm.at[pl.ds(t * BE, BE)],
                         wijb[slot], semw.at[slot])

    def _wait_side(slot, t):
        pltpu.make_async_copy(side_hbm.at[pl.ds(t * SIDE, SIDE)],
                              sideb[slot], sems.at[slot]).wait()

    def _wait_wij(slot, t):
        pltpu.make_async_copy(wij_hbm.at[pl.ds(t * BE, BE)],
                              wijb[slot], semw.at[slot]).wait()

    def _issue_gather(slot):
        pltpu.async_copy(t_hbm.at[sideb[slot].at[pl.ds(BE, BE)]],
                         trowsb[slot], semg.at[slot])

    def _wait_gather(slot):
        pltpu.make_async_copy(t_hbm.at[sideb[slot].at[pl.ds(BE, BE)]],
                              trowsb[slot], semg.at[slot]).wait()

    def _compute(slot, node_base):
        side_v = sideb[slot]
        wij_v = wijb[slot]
        trows_v = trowsb[slot]
        for vc in range(BE // 16):
            vb = vc * 16
            iv = side_v[pl.ds(vb, 16)]
            valid = (iv >= node_base) & (iv < node_base + nw)
            loc16 = jnp.where(valid, iv - node_base, nw)
            d016 = plsc.bitcast(side_v[pl.ds(2 * BE + vb, 16)], jnp.float32)
            d116 = plsc.bitcast(side_v[pl.ds(3 * BE + vb, 16)], jnp.float32)
            d216 = plsc.bitcast(side_v[pl.ds(4 * BE + vb, 16)], jnp.float32)

            def _edge4(i, _):
                for eu in range(2):
                    e16 = i * 2 + eu
                    e = vb + e16
                    locb = _bcast(loc16, e16)
                    d0 = _bcast(d016, e16)
                    d1 = _bcast(d116, e16)
                    d2 = _bcast(d216, e16)
                    vrow = trows_v[e, pl.ds(384, 16)]
                    v0 = _bcast(vrow, 0)
                    v1 = _bcast(vrow, 1)
                    v2 = _bcast(vrow, 2)
                    dd = (d0, d1, d2)
                    vv = (v0, v1, v2)
                    for f in range(F // 16):
                        s = f * 16
                        xv = trows_v[e, pl.ds(s, 16)] * wij_v[e, pl.ds(s, 16)]
                        plsc.addupdate_scatter(acc_v, [locb, cols + s], xv)
                        t1v = trows_v[e, pl.ds(128 + s, 16)] * wij_v[e, pl.ds(128 + s, 16)]
                        t2v = trows_v[e, pl.ds(256 + s, 16)] * wij_v[e, pl.ds(256 + s, 16)]
                        for c in range(3):
                            mv = t1v * dd[c] + t2v * vv[c]
                            plsc.addupdate_scatter(
                                acc_v, [locb, cols + (128 * (c + 1) + s)], mv)
                return 0

            lax.fori_loop(0, 8, _edge4, 0)

    for k in range(RPW):
        node_base = (w * RPW + k) * nw
        e_start = bw[k]
        e_end = bw[k + 1]

        def _zr(r, _):
            for cc in range(512 // 16):
                acc_v[r, pl.ds(cc * 16, 16)] = zero16
            return 0

        lax.fori_loop(0, acc_rows, _zr, 0)

        t0 = e_start // BE
        t1 = (e_end + BE - 1) // BE
        nb = t1 - t0

        @pl.when(nb > 0)
        def _():
            _issue_sw(0, t0)
            _wait_side(0, t0)
            _issue_gather(0)

        @pl.when(nb > 1)
        def _():
            _issue_sw(1, t0 + 1)

        def _pair(p, _):
            i0 = 2 * p
            i1 = 2 * p + 1

            @pl.when(i1 < nb)
            def _():
                _wait_side(1, t0 + i1)
                _issue_gather(1)

            _wait_gather(0)
            _wait_wij(0, t0 + i0)
            _compute(0, node_base)

            @pl.when(i0 + 2 < nb)
            def _():
                _issue_sw(0, t0 + i0 + 2)
                _wait_side(0, t0 + i0 + 2)
                _issue_gather(0)

            @pl.when(i1 < nb)
            def _():
                _wait_gather(1)
                _wait_wij(1, t0 + i1)
                _compute(1, node_base)

                @pl.when(i1 + 2 < nb)
                def _():
                    _issue_sw(1, t0 + i1 + 2)

            return 0

        lax.fori_loop(0, (nb + 1) // 2, _pair, 0)

        pltpu.async_copy(acc_v.at[pl.ds(0, nw)],
                         out_hbm.at[pl.ds(node_base, nw)], semo).wait()


def _edge_phase(t_tab, wij2, side, bt, nw):
    n_pad = NWORK * RPW * nw
    mesh = plsc.VectorSubcoreMesh(core_axis_name="c", subcore_axis_name="s")
    body = functools.partial(_edge_body, nw)
    f = pl.kernel(
        body,
        out_type=jax.ShapeDtypeStruct((n_pad, 512), jnp.float32),
        mesh=mesh,
        scratch_types=[
            pltpu.VMEM((SIDE,), jnp.int32),
            pltpu.VMEM((SIDE,), jnp.int32),
            pltpu.VMEM((BE, 384), jnp.float32),
            pltpu.VMEM((BE, 384), jnp.float32),
            pltpu.VMEM((BE, TCOLS), jnp.float32),
            pltpu.VMEM((BE, TCOLS), jnp.float32),
            pltpu.VMEM((NWORK, 16), jnp.int32),
            pltpu.VMEM((nw + 8, 512), jnp.float32),
            pltpu.SemaphoreType.DMA((2,)),
            pltpu.SemaphoreType.DMA((2,)),
            pltpu.SemaphoreType.DMA((2,)),
            pltpu.SemaphoreType.DMA,
        ],
        compiler_params=pltpu.CompilerParams(needs_layout_passes=False),
    )
    return f(t_tab, wij2, side, bt)


def _post_body(q_ref, acc_ref, wv_ref, rw1_ref, rb1_ref, rwt_ref, rbt_ref,
               qo_ref, muo_ref, dtm_ref):
    dq = acc_ref[:, 0:128]
    qo_ref[...] = q_ref[...] + dq
    m = []
    for c in range(3):
        mc = acc_ref[:, 128 * (c + 1):128 * (c + 2)] + wv_ref[:, 128 * c:128 * (c + 1)]
        muo_ref[:, 128 * c:128 * (c + 1)] = mc
        m.append(mc)
    si = jnp.sqrt(m[0] * m[0] + m[1] * m[1] + m[2] * m[2])
    h = jnp.maximum(si @ rw1_ref[...] + rb1_ref[...], 0.0)
    bn = h.shape[0]
    pairs = [(0, 0), (0, 1), (0, 2), (1, 1), (1, 2), (2, 2)]
    r2 = jnp.concatenate([m[i] * m[j] for i, j in pairs], axis=0)
    gm = r2 @ rwt_ref[...]
    dall = r2 @ rbt_ref[...]
    h6 = jnp.concatenate([h] * 6, axis=0)
    for k in range(32):
        dall = dall + h6[:, k:k + 1] * gm[:, 128 * k:128 * (k + 1)]
    for p, (i, j) in enumerate(pairs):
        d = dall[p * bn:(p + 1) * bn, :]
        dtm_ref[:, 3 * i + j, :] = d
        if i != j:
            dtm_ref[:, 3 * j + i, :] = d


def _post_phase(q2, acc, wv, Rw1, Rb1, Rw2, Rb2):
    n = q2.shape[0]
    bn = 200
    grid = n // bn
    rwt = Rw2.reshape(32, F, F).transpose(2, 0, 1).reshape(F, 32 * F)
    rbt = Rb2.reshape(F, F).T
    full = lambda shape: pl.BlockSpec(shape, lambda i: (0, 0))
    return pl.pallas_call(
        _post_body,
        grid=(grid,),
        in_specs=[
            pl.BlockSpec((bn, F), lambda i: (i, 0)),
            pl.BlockSpec((bn, 512), lambda i: (i, 0)),
            pl.BlockSpec((bn, 384), lambda i: (i, 0)),
            full((F, 32)), full((1, 32)), full((F, 32 * F)), full((F, F)),
        ],
        out_specs=[
            pl.BlockSpec((bn, F), lambda i: (i, 0)),
            pl.BlockSpec((bn, 384), lambda i: (i, 0)),
            pl.BlockSpec((bn, 9, F), lambda i: (i, 0, 0)),
        ],
        out_shape=[
            jax.ShapeDtypeStruct((n, F), jnp.float32),
            jax.ShapeDtypeStruct((n, 384), jnp.float32),
            jax.ShapeDtypeStruct((n, 9, F), jnp.float32),
        ],
    )(q2, acc, wv, Rw1, Rb1.reshape(1, -1), rwt, rbt)


def kernel(q, mu, Wij, dir_ij, idx_i, idx_j, n_atoms, W1, b1, W2, b2,
           Cw1, Cb1, Cw2, Cb2, Rw1, Rb1, Rw2, Rb2):
    n = q.shape[0]
    e_total = idx_i.shape[0]
    q2 = q.reshape(n, F)
    mu2 = mu.reshape(n, 768)
    wij2 = Wij.reshape(e_total, 384)

    dbits = lax.bitcast_convert_type(dir_ij.T, jnp.int32)
    side = jnp.concatenate([idx_i.reshape(1, -1), idx_j.reshape(1, -1), dbits], axis=0)
    side = side.reshape(5, e_total // BE, BE).transpose(1, 0, 2).reshape(-1)

    nranges = NWORK * RPW
    nw = ((-(-n // nranges) + 7) // 8) * 8
    marks = jnp.arange(nranges + 1, dtype=jnp.int32) * nw
    ss = jnp.searchsorted(idx_i, marks).astype(jnp.int32)
    sel = jnp.arange(NWORK)[:, None] * RPW + jnp.arange(RPW + 1)[None, :]
    bt = jnp.zeros((NWORK, 16), jnp.int32)
    bt = bt.at[:, :RPW + 1].set(ss[sel])

    t_tab, wv = _node_table(q2, mu2, W1, b1, W2, b2, Cw1, Cb1, Cw2, Cb2)
    acc = _edge_phase(t_tab, wij2, side, bt, nw)
    q_out, mu_new, dtm9 = _post_phase(q2, acc, wv, Rw1, Rb1, Rw2, Rb2)
    return (q_out.reshape(n, 1, F),
            mu_new.reshape(n, 3, F),
            dtm9.transpose(0, 2, 1).reshape(n, F, 3, 3))


# speedup vs baseline: 1.7552x; 1.0001x over previous
import functools

import jax
import jax.numpy as jnp
from jax import lax
from jax.experimental import pallas as pl
from jax.experimental.pallas import tpu as pltpu
from jax.experimental.pallas import tpu_sc as plsc

F = 128
TCOLS = 512
BE = 32
SIDE = 5 * BE
RPW = 4
NWORK = 32


def _node_table_body(q_ref, mu_ref, w1_ref, b1_ref, w2_ref, b2_ref,
                     cw1_ref, cb1_ref, cw2_ref, cb2_ref, t_ref, wv_ref):
    qb = q_ref[...]
    h = qb @ w1_ref[...] + b1_ref[...]
    h = h / (1.0 + jnp.exp(-h))
    x = h @ w2_ref[...] + b2_ref[...]
    mu0 = mu_ref[:, 0:256]
    mu1 = mu_ref[:, 256:512]
    mu2 = mu_ref[:, 512:768]
    si = jnp.sqrt(mu0 * mu0 + mu1 * mu1 + mu2 * mu2)
    hh = jnp.maximum(si @ cw1_ref[...] + cb1_ref[...], 0.0)
    logits = hh @ cw2_ref[...] + cb2_ref[...]
    m = jnp.max(logits, axis=-1, keepdims=True)
    e = jnp.exp(logits - m)
    w = e / jnp.sum(e, axis=-1, keepdims=True)
    bn = qb.shape[0]
    v0 = jnp.sum(mu0, axis=-1, keepdims=True)
    v1 = jnp.sum(mu1, axis=-1, keepdims=True)
    v2 = jnp.sum(mu2, axis=-1, keepdims=True)
    t_ref[:, 0:256] = x[:, 0:256]
    t_ref[:, 256:384] = x[:, 256:384] * w
    t_ref[:, 384:512] = jnp.concatenate(
        [v0, v1, v2, jnp.zeros((bn, 125), jnp.float32)], axis=-1)
    wv_ref[:, 0:128] = w * v0
    wv_ref[:, 128:256] = w * v1
    wv_ref[:, 256:384] = w * v2


def _node_table(q2, mu2, W1, b1, W2, b2, Cw1, Cb1, Cw2, Cb2):
    n = q2.shape[0]
    bn = 400
    grid = n // bn
    full = lambda shape: pl.BlockSpec(shape, lambda i: (0, 0))
    return pl.pallas_call(
        _node_table_body,
        grid=(grid,),
        in_specs=[
            pl.BlockSpec((bn, F), lambda i: (i, 0)),
            pl.BlockSpec((bn, 768), lambda i: (i, 0)),
            full((F, F)), full((1, F)), full((F, 384)), full((1, 384)),
            full((256, 32)), full((1, 32)), full((32, F)), full((1, F)),
        ],
        out_specs=[pl.BlockSpec((bn, TCOLS), lambda i: (i, 0)),
                   pl.BlockSpec((bn, 384), lambda i: (i, 0))],
        out_shape=[jax.ShapeDtypeStruct((n, TCOLS), jnp.float32),
                   jax.ShapeDtypeStruct((n, 384), jnp.float32)],
    )(q2, mu2, W1, b1.reshape(1, -1), W2, b2.reshape(1, -1),
      Cw1, Cb1.reshape(1, -1), Cw2, Cb2.reshape(1, -1))


def _edge_body(nw,
               t_hbm, wij_hbm, side_hbm, bt_hbm, out_hbm,
               side0_v, side1_v, wij0_v, wij1_v, trows0_v, trows1_v,
               bt_v, acc_v, sems, semw, semg, semo):
    cid = lax.axis_index("c")
    sid = lax.axis_index("s")
    w = cid * 16 + sid
    pltpu.sync_copy(bt_hbm, bt_v)
    bw = bt_v[w, pl.ds(0, 16)]
    cols = lax.iota(jnp.int32, 16)
    zero16 = jnp.zeros((16,), jnp.float32)
    acc_rows = acc_v.shape[0]
    sideb = (side0_v, side1_v)
    wijb = (wij0_v, wij1_v)
    trowsb = (trows0_v, trows1_v)

    def _bcast(v, lane):
        idx = jnp.full((16,), lane, jnp.int32)
        return v.at[idx].get(mode=lax.GatherScatterMode.PROMISE_IN_BOUNDS)

    def _issue_sw(slot, t):
        pltpu.async_copy(side_hbm.at[pl.ds(t * SIDE, SIDE)],
                         sideb[slot], sems.at[slot])
        pltpu.async_copy(wij_hbm.at[pl.ds(t * BE, BE)],
                         wijb[slot], semw.at[slot])

    def _wait_side(slot, t):
        pltpu.make_async_copy(side_hbm.at[pl.ds(t * SIDE, SIDE)],
                              sideb[slot], sems.at[slot]).wait()

    def _wait_wij(slot, t):
        pltpu.make_async_copy(wij_hbm.at[pl.ds(t * BE, BE)],
                              wijb[slot], semw.at[slot]).wait()

    def _issue_gather(slot):
        pltpu.async_copy(t_hbm.at[sideb[slot].at[pl.ds(BE, BE)]],
                         trowsb[slot], semg.at[slot])

    def _wait_gather(slot):
        pltpu.make_async_copy(t_hbm.at[sideb[slot].at[pl.ds(BE, BE)]],
                              trowsb[slot], semg.at[slot]).wait()

    def _compute(slot, node_base):
        side_v = sideb[slot]
        wij_v = wijb[slot]
        trows_v = trowsb[slot]
        for vc in range(BE // 16):
            vb = vc * 16
            iv = side_v[pl.ds(vb, 16)]
            valid = (iv >= node_base) & (iv < node_base + nw)
            loc16 = jnp.where(valid, iv - node_base, nw)
            d016 = plsc.bitcast(side_v[pl.ds(2 * BE + vb, 16)], jnp.float32)
            d116 = plsc.bitcast(side_v[pl.ds(3 * BE + vb, 16)], jnp.float32)
            d216 = plsc.bitcast(side_v[pl.ds(4 * BE + vb, 16)], jnp.float32)

            def _edge4(i, _):
                for eu in range(2):
                    e16 = i * 2 + eu
                    e = vb + e16
                    d0 = _bcast(d016, e16)
                    d1 = _bcast(d116, e16)
                    d2 = _bcast(d216, e16)
                    vrow = trows_v[e, pl.ds(384, 16)]
                    v0 = _bcast(vrow, 0)
                    v1 = _bcast(vrow, 1)
                    v2 = _bcast(vrow, 2)
                    dd = (d0, d1, d2)
                    vv = (v0, v1, v2)
                    locb = _bcast(loc16, e16)
                    for f in range(F // 16):
                        s = f * 16
                        xv = trows_v[e, pl.ds(s, 16)] * wij_v[e, pl.ds(s, 16)]
                        plsc.addupdate_scatter(acc_v, [locb, cols + s], xv)
                        t1v = trows_v[e, pl.ds(128 + s, 16)] * wij_v[e, pl.ds(128 + s, 16)]
                        t2v = trows_v[e, pl.ds(256 + s, 16)] * wij_v[e, pl.ds(256 + s, 16)]
                        for c in range(3):
                            mv = t1v * dd[c] + t2v * vv[c]
                            plsc.addupdate_scatter(
                                acc_v, [locb, cols + (128 * (c + 1) + s)], mv)
                return 0

            lax.fori_loop(0, 8, _edge4, 0)

    for k in range(RPW):
        node_base = (w * RPW + k) * nw
        e_start = bw[k]
        e_end = bw[k + 1]

        def _zr(r, _):
            for cc in range(512 // 16):
                acc_v[r, pl.ds(cc * 16, 16)] = zero16
            return 0

        lax.fori_loop(0, acc_rows, _zr, 0)

        t0 = e_start // BE
        t1 = (e_end + BE - 1) // BE
        nb = t1 - t0

        @pl.when(nb > 0)
        def _():
            _issue_sw(0, t0)
            _wait_side(0, t0)
            _issue_gather(0)

        @pl.when(nb > 1)
        def _():
            _issue_sw(1, t0 + 1)

        def _pair(p, _):
            i0 = 2 * p
            i1 = 2 * p + 1

            @pl.when(i1 < nb)
            def _():
                _wait_side(1, t0 + i1)
                _issue_gather(1)

            _wait_gather(0)
            _wait_wij(0, t0 + i0)
            _compute(0, node_base)

            @pl.when(i0 + 2 < nb)
            def _():
                _issue_sw(0, t0 + i0 + 2)
                _wait_side(0, t0 + i0 + 2)
                _issue_gather(0)

            @pl.when(i1 < nb)
            def _():
                _wait_gather(1)
                _wait_wij(1, t0 + i1)
                _compute(1, node_base)

                @pl.when(i1 + 2 < nb)
                def _():
                    _issue_sw(1, t0 + i1 + 2)

            return 0

        lax.fori_loop(0, (nb + 1) // 2, _pair, 0)

        pltpu.async_copy(acc_v.at[pl.ds(0, nw)],
                         out_hbm.at[pl.ds(node_base, nw)], semo).wait()


def _edge_phase(t_tab, wij2, side, bt, nw):
    n_pad = NWORK * RPW * nw
    mesh = plsc.VectorSubcoreMesh(core_axis_name="c", subcore_axis_name="s")
    body = functools.partial(_edge_body, nw)
    f = pl.kernel(
        body,
        out_type=jax.ShapeDtypeStruct((n_pad, 512), jnp.float32),
        mesh=mesh,
        scratch_types=[
            pltpu.VMEM((SIDE,), jnp.int32),
            pltpu.VMEM((SIDE,), jnp.int32),
            pltpu.VMEM((BE, 384), jnp.float32),
            pltpu.VMEM((BE, 384), jnp.float32),
            pltpu.VMEM((BE, TCOLS), jnp.float32),
            pltpu.VMEM((BE, TCOLS), jnp.float32),
            pltpu.VMEM((NWORK, 16), jnp.int32),
            pltpu.VMEM((nw + 8, 512), jnp.float32),
            pltpu.SemaphoreType.DMA((2,)),
            pltpu.SemaphoreType.DMA((2,)),
            pltpu.SemaphoreType.DMA((2,)),
            pltpu.SemaphoreType.DMA,
        ],
        compiler_params=pltpu.CompilerParams(needs_layout_passes=False, use_tc_tiling_on_sc=True),
    )
    return f(t_tab, wij2, side, bt)


def _post_body(q_ref, acc_ref, wv_ref, rw1_ref, rb1_ref, rwt_ref, rbt_ref,
               qo_ref, muo_ref, dtm_ref):
    dq = acc_ref[:, 0:128]
    qo_ref[...] = q_ref[...] + dq
    m = []
    for c in range(3):
        mc = acc_ref[:, 128 * (c + 1):128 * (c + 2)] + wv_ref[:, 128 * c:128 * (c + 1)]
        muo_ref[:, 128 * c:128 * (c + 1)] = mc
        m.append(mc)
    si = jnp.sqrt(m[0] * m[0] + m[1] * m[1] + m[2] * m[2])
    h = jnp.maximum(si @ rw1_ref[...] + rb1_ref[...], 0.0)
    bn = h.shape[0]
    pairs = [(0, 0), (0, 1), (0, 2), (1, 1), (1, 2), (2, 2)]
    r2 = jnp.concatenate([m[i] * m[j] for i, j in pairs], axis=0)
    gm = r2 @ rwt_ref[...]
    dall = r2 @ rbt_ref[...]
    h6 = jnp.concatenate([h] * 6, axis=0)
    for k in range(32):
        dall = dall + h6[:, k:k + 1] * gm[:, 128 * k:128 * (k + 1)]
    for p, (i, j) in enumerate(pairs):
        d = dall[p * bn:(p + 1) * bn, :]
        dtm_ref[:, 3 * i + j, :] = d
        if i != j:
            dtm_ref[:, 3 * j + i, :] = d


def _post_phase(q2, acc, wv, Rw1, Rb1, Rw2, Rb2):
    n = q2.shape[0]
    bn = 200
    grid = n // bn
    rwt = Rw2.reshape(32, F, F).transpose(2, 0, 1).reshape(F, 32 * F)
    rbt = Rb2.reshape(F, F).T
    full = lambda shape: pl.BlockSpec(shape, lambda i: (0, 0))
    return pl.pallas_call(
        _post_body,
        grid=(grid,),
        in_specs=[
            pl.BlockSpec((bn, F), lambda i: (i, 0)),
            pl.BlockSpec((bn, 512), lambda i: (i, 0)),
            pl.BlockSpec((bn, 384), lambda i: (i, 0)),
            full((F, 32)), full((1, 32)), full((F, 32 * F)), full((F, F)),
        ],
        out_specs=[
            pl.BlockSpec((bn, F), lambda i: (i, 0)),
            pl.BlockSpec((bn, 384), lambda i: (i, 0)),
            pl.BlockSpec((bn, 9, F), lambda i: (i, 0, 0)),
        ],
        out_shape=[
            jax.ShapeDtypeStruct((n, F), jnp.float32),
            jax.ShapeDtypeStruct((n, 384), jnp.float32),
            jax.ShapeDtypeStruct((n, 9, F), jnp.float32),
        ],
    )(q2, acc, wv, Rw1, Rb1.reshape(1, -1), rwt, rbt)


def kernel(q, mu, Wij, dir_ij, idx_i, idx_j, n_atoms, W1, b1, W2, b2,
           Cw1, Cb1, Cw2, Cb2, Rw1, Rb1, Rw2, Rb2):
    n = q.shape[0]
    e_total = idx_i.shape[0]
    q2 = q.reshape(n, F)
    mu2 = mu.reshape(n, 768)
    wij2 = Wij.reshape(e_total, 384)

    dbits = lax.bitcast_convert_type(dir_ij.T, jnp.int32)
    side = jnp.concatenate([idx_i.reshape(1, -1), idx_j.reshape(1, -1), dbits], axis=0)
    side = side.reshape(5, e_total // BE, BE).transpose(1, 0, 2).reshape(-1)

    nranges = NWORK * RPW
    nw = ((-(-n // nranges) + 7) // 8) * 8
    marks = jnp.arange(nranges + 1, dtype=jnp.int32) * nw
    ss = jnp.searchsorted(idx_i, marks).astype(jnp.int32)
    sel = jnp.arange(NWORK)[:, None] * RPW + jnp.arange(RPW + 1)[None, :]
    bt = jnp.zeros((NWORK, 16), jnp.int32)
    bt = bt.at[:, :RPW + 1].set(ss[sel])

    t_tab, wv = _node_table(q2, mu2, W1, b1, W2, b2, Cw1, Cb1, Cw2, Cb2)
    acc = _edge_phase(t_tab, wij2, side, bt, nw)
    q_out, mu_new, dtm9 = _post_phase(q2, acc, wv, Rw1, Rb1, Rw2, Rb2)
    return (q_out.reshape(n, 1, F),
            mu_new.reshape(n, 3, F),
            dtm9.transpose(0, 2, 1).reshape(n, F, 3, 3))
